# Initial kernel scaffold; baseline (speedup 1.0000x reference)
import jax
import jax.numpy as jnp
from jax.experimental import pallas as pl


def kernel(x, emb_data, tables):
    raise NotImplementedError("write your pallas kernel here")


# speedup vs baseline: 3.9857x; 3.9857x over previous
import jax
import jax.numpy as jnp
from jax import lax
from jax.experimental import pallas as pl
from jax.experimental.pallas import tpu as pltpu
from jax.experimental.pallas import tpu_sc as plsc

N_COLS = 26
N_CAT = 1000
DIM = 77
DIM_PAD = 80
B = 16384
D_DENSE = 13
D_OUT = D_DENSE + N_COLS * DIM

NC = 2
NS = 16
NW = NC * NS
ROWS_PER_W = B // NW
CB = 16
NCHUNK = ROWS_PER_W // CB

_SEG_OFFS = (0, 16, 32, 48, DIM - 16)


def _sc_body(x1_hbm, embT_hbm, tabf_hbm, out1_hbm, idx_v, gall_v, xbuf_v,
             sbuf_v, sem_ld, sem_g, sem_w):
    wid = lax.axis_index("s") * NC + lax.axis_index("c")
    wbase = wid * ROWS_PER_W

    def ld_descs(ci):
        sl = lax.rem(ci, 2)
        sl3 = lax.rem(ci, 3)
        base = wbase + ci * CB
        return (
            pltpu.make_async_copy(embT_hbm.at[:, pl.ds(base, CB)],
                                  idx_v.at[sl], sem_ld),
            pltpu.make_async_copy(
                x1_hbm.at[pl.ds(base * D_DENSE, CB * D_DENSE)],
                xbuf_v.at[sl3, pl.ds(0, CB * D_DENSE)], sem_ld),
        )

    def fire_loads(ci):
        for d in ld_descs(ci):
            d.start()

    def wait_loads(ci):
        for d in ld_descs(ci):
            d.wait()

    def fire_gathers(ci):
        sl = lax.rem(ci, 2)

        def col(i, c2):
            idx_v[sl, i, :] = idx_v[sl, i, :] + i * N_CAT
            pltpu.make_async_copy(tabf_hbm.at[idx_v.at[sl, i]],
                                  gall_v.at[sl, i], sem_g).start()
            return c2

        lax.fori_loop(0, N_COLS, col, 0)

    def drain_gathers(ci):
        sl = lax.rem(ci, 2)

        def col(i, c2):
            pltpu.make_async_copy(tabf_hbm.at[idx_v.at[sl, i]],
                                  gall_v.at[sl, i], sem_g).wait()
            return c2

        lax.fori_loop(0, N_COLS, col, 0)

    def wr_desc(ci):
        base = wbase + ci * CB
        return pltpu.make_async_copy(
            sbuf_v, out1_hbm.at[pl.ds(base * D_OUT, CB * D_OUT)], sem_w)

    def assemble(ci):
        sl = lax.rem(ci, 2)

        def row_asm(r, c2):
            b0 = r * D_OUT
            sbuf_v[pl.ds(b0, 16)] = xbuf_v[lax.rem(ci, 3), pl.ds(r * D_DENSE, 16)]
            for i0 in range(0, N_COLS, 2):
                vals = [gall_v[sl, i, r, pl.ds(o, 16)]
                        for i in (i0, i0 + 1) for o in _SEG_OFFS]
                k = 0
                for i in (i0, i0 + 1):
                    for o in _SEG_OFFS:
                        sbuf_v[pl.ds(b0 + D_DENSE + i * DIM + o, 16)] = vals[k]
                        k += 1
            return c2

        lax.fori_loop(0, CB, row_asm, 0)

    fire_loads(0)
    wait_loads(0)
    fire_gathers(0)
    fire_loads(1)

    def body(ci, carry):
        drain_gathers(ci)

        @pl.when(ci + 1 < NCHUNK)
        def _():
            wait_loads(ci + 1)
            fire_gathers(ci + 1)

        @pl.when(ci + 2 < NCHUNK)
        def _():
            fire_loads(ci + 2)

        @pl.when(ci > 0)
        def _():
            wr_desc(ci - 1).wait()

        assemble(ci)
        wr_desc(ci).start()
        return carry

    lax.fori_loop(0, NCHUNK, body, 0)
    wr_desc(NCHUNK - 1).wait()


@jax.jit
def _sc_concat_embed(x1, embT, tabf):
    mesh = plsc.VectorSubcoreMesh(core_axis_name="c", subcore_axis_name="s",
                                  num_cores=NC, num_subcores=NS)
    return pl.kernel(
        _sc_body,
        out_type=jax.ShapeDtypeStruct((B * D_OUT,), jnp.float32),
        mesh=mesh,
        scratch_types=[
            pltpu.VMEM((2, N_COLS, CB), jnp.int32),
            pltpu.VMEM((2, N_COLS, CB, DIM_PAD), jnp.float32),
            pltpu.VMEM((3, CB * D_DENSE + 16), jnp.float32),
            pltpu.VMEM((CB * D_OUT,), jnp.float32),
            pltpu.SemaphoreType.DMA,
            pltpu.SemaphoreType.DMA,
            pltpu.SemaphoreType.DMA,
        ],
        compiler_params=pltpu.CompilerParams(use_tc_tiling_on_sc=False),
    )(x1, embT, tabf)


def kernel(x, emb_data, tables):
    embT = emb_data.T
    tabf = jnp.pad(tables.reshape(N_COLS * N_CAT, DIM),
                   ((0, 0), (0, DIM_PAD - DIM)))
    out1 = _sc_concat_embed(x.reshape(B * D_DENSE), embT, tabf)
    return out1.reshape(B, D_OUT)
